# baseline (speedup 1.0000x reference)
import functools

import jax
import jax.numpy as jnp
from jax import lax
from jax.experimental import pallas as pl
from jax.experimental.pallas import tpu as pltpu
from jax.experimental.pallas import tpu_sc as plsc

N = 10000
E = 320000
D = 128
NPAD = 10240
NC = 2
NS = 16
NW = NC * NS
EPT = E // NW
CH = 80
CPT = EPT // CH
NBUF = 5
NWAVE = CPT // NBUF
RPT = NPAD // NS

CH3 = 40
CPT3 = EPT // CH3
NWAVE3 = CPT3 // NBUF


def _sc_mesh():
    return plsc.VectorSubcoreMesh(core_axis_name="c", subcore_axis_name="s")


def _deg_kernel(dst_hbm, out_hbm, didx_v, ones_v, bounce_v, ssem, isem, deg_sh):
    c = lax.axis_index("c")
    s = lax.axis_index("s")
    wid = s * NC + c

    for j in range(RPT // 16):
        bounce_v[pl.ds(j * 16, 16)] = jnp.zeros((16,), jnp.float32)
    pltpu.sync_copy(bounce_v, deg_sh.at[pl.ds(s * RPT, RPT)])
    for j in range(CH // 16):
        ones_v[pl.ds(j * 16, 16)] = jnp.ones((16,), jnp.float32)
    plsc.subcore_barrier()

    base = wid * EPT

    def wait_sc(j):
        pltpu.make_async_copy(out_hbm.at[0, pl.ds(0, CH)], ones_v, ssem.at[j]).wait()

    def wait_ix(j):
        pltpu.make_async_copy(
            dst_hbm.at[pl.ds(0, CH)], didx_v.at[j], isem.at[j]
        ).wait()

    for j in range(NBUF):
        pltpu.async_copy(
            dst_hbm.at[pl.ds(base + j * CH, CH)], didx_v.at[j], isem.at[j]
        )

    def wave(g, carry):
        for j in range(NBUF):
            wait_ix(j)
            pltpu.async_copy(ones_v, deg_sh.at[didx_v.at[j]], ssem.at[j], add=True)
        for j in range(NBUF):
            t = (g + 1) * NBUF + j
            wait_sc(j)
            pltpu.async_copy(
                dst_hbm.at[pl.ds(base + t * CH, CH)], didx_v.at[j], isem.at[j]
            )
        return carry

    lax.fori_loop(0, NWAVE - 1, wave, 0)
    for j in range(NBUF):
        wait_ix(j)
        pltpu.async_copy(ones_v, deg_sh.at[didx_v.at[j]], ssem.at[j], add=True)
    for j in range(NBUF):
        wait_sc(j)
    plsc.subcore_barrier()

    pltpu.sync_copy(deg_sh.at[pl.ds(s * RPT, RPT)], out_hbm.at[c, pl.ds(s * RPT, RPT)])


def _degree_partials(dst):
    k = functools.partial(
        pl.kernel,
        out_type=jax.ShapeDtypeStruct((NC, NPAD), jnp.float32),
        mesh=_sc_mesh(),
        scratch_types=[
            pltpu.VMEM((NBUF, CH), jnp.int32),
            pltpu.VMEM((CH,), jnp.float32),
            pltpu.VMEM((RPT,), jnp.float32),
            pltpu.SemaphoreType.DMA((NBUF,)),
            pltpu.SemaphoreType.DMA((NBUF,)),
            pltpu.VMEM_SHARED((NPAD,), jnp.float32),
        ],
    )(_deg_kernel)
    return k(dst)


def _h2_body(x_ref, w_ref, degT_ref, h2_ref):
    dsum = degT_ref[:, 0:1] + degT_ref[:, 1:2] + 1.0
    dinv = lax.rsqrt(dsum)
    h = jnp.dot(x_ref[...], w_ref[...], preferred_element_type=jnp.float32)
    h2_ref[...] = h * dinv


def _h2_compute(x, W, degT):
    nb = 10
    rb = N // nb
    return pl.pallas_call(
        _h2_body,
        grid=(nb,),
        in_specs=[
            pl.BlockSpec((rb, D), lambda i: (i, 0)),
            pl.BlockSpec((D, D), lambda i: (0, 0)),
            pl.BlockSpec((rb, NC), lambda i: (i, 0)),
        ],
        out_specs=pl.BlockSpec((rb, D), lambda i: (i, 0)),
        out_shape=jax.ShapeDtypeStruct((N, D), jnp.float32),
    )(x, W, degT)


def _msg_kernel(src_hbm, dst_hbm, h2_hbm, out_hbm,
                sidx_v, didx_v, rows_v, gsem, ssem, isem, zsem, acc_sh):
    c = lax.axis_index("c")
    s = lax.axis_index("s")
    wid = s * NC + c

    def zrow(i, carry):
        for j in range(D // 16):
            rows_v[0, i, pl.ds(j * 16, 16)] = jnp.zeros((16,), jnp.float32)
        return carry

    lax.fori_loop(0, CH3, zrow, 0)

    def drain_z(k):
        for _ in range(k):
            pltpu.make_async_copy(
                h2_hbm.at[pl.ds(0, CH3)], rows_v.at[0], zsem
            ).wait()

    @pl.when(c == 0)
    def _():
        @pl.when(s < NS - 1)
        def _():
            pltpu.sync_copy(h2_hbm.at[pl.ds(s * RPT, RPT)],
                            acc_sh.at[pl.ds(s * RPT, RPT)])

        @pl.when(s == NS - 1)
        def _():
            pltpu.sync_copy(h2_hbm.at[pl.ds((NS - 1) * RPT, N - (NS - 1) * RPT)],
                            acc_sh.at[pl.ds((NS - 1) * RPT, N - (NS - 1) * RPT)])
            for v in range((NPAD - N) // CH3):
                pltpu.async_copy(rows_v.at[0],
                                 acc_sh.at[pl.ds(N + v * CH3, CH3)], zsem)
            drain_z((NPAD - N) // CH3)

    @pl.when(c == 1)
    def _():
        for v in range(RPT // CH3):
            pltpu.async_copy(rows_v.at[0],
                             acc_sh.at[pl.ds(s * RPT + v * CH3, CH3)], zsem)
        drain_z(RPT // CH3)

    plsc.subcore_barrier()

    pltpu.sync_copy(src_hbm.at[pl.ds(wid * EPT, EPT)], sidx_v)
    base = wid * EPT

    def wait_rows(sem_ref, j):
        pltpu.make_async_copy(h2_hbm.at[pl.ds(0, CH3)], rows_v.at[j], sem_ref.at[j]).wait()

    def wait_idx(j):
        pltpu.make_async_copy(
            dst_hbm.at[pl.ds(0, CH3)], didx_v.at[j], isem.at[j]
        ).wait()

    for j in range(NBUF):
        pltpu.async_copy(dst_hbm.at[pl.ds(base + j * CH3, CH3)],
                         didx_v.at[j], isem.at[j])
        pltpu.async_copy(h2_hbm.at[sidx_v.at[pl.ds(j * CH3, CH3)]],
                         rows_v.at[j], gsem.at[j])

    def wave(g, carry):
        for j in range(NBUF):
            wait_rows(gsem, j)
            wait_idx(j)
            pltpu.async_copy(rows_v.at[j], acc_sh.at[didx_v.at[j]],
                             ssem.at[j], add=True)
        for j in range(NBUF):
            t = (g + 1) * NBUF + j
            wait_rows(ssem, j)
            pltpu.async_copy(dst_hbm.at[pl.ds(base + t * CH3, CH3)],
                             didx_v.at[j], isem.at[j])
            pltpu.async_copy(h2_hbm.at[sidx_v.at[pl.ds(t * CH3, CH3)]],
                             rows_v.at[j], gsem.at[j])
        return carry

    lax.fori_loop(0, NWAVE3 - 1, wave, 0)
    for j in range(NBUF):
        wait_rows(gsem, j)
        wait_idx(j)
        pltpu.async_copy(rows_v.at[j], acc_sh.at[didx_v.at[j]],
                         ssem.at[j], add=True)
    for j in range(NBUF):
        wait_rows(ssem, j)
    plsc.subcore_barrier()

    pltpu.sync_copy(acc_sh.at[pl.ds(s * RPT, RPT)],
                    out_hbm.at[c, pl.ds(s * RPT, RPT)])


def _message_partials(src, dst, h2):
    k = functools.partial(
        pl.kernel,
        out_type=jax.ShapeDtypeStruct((NC, NPAD, D), jnp.float32),
        mesh=_sc_mesh(),
        scratch_types=[
            pltpu.VMEM((EPT,), jnp.int32),
            pltpu.VMEM((NBUF, CH3), jnp.int32),
            pltpu.VMEM((NBUF, CH3, D), jnp.float32),
            pltpu.SemaphoreType.DMA((NBUF,)),
            pltpu.SemaphoreType.DMA((NBUF,)),
            pltpu.SemaphoreType.DMA((NBUF,)),
            pltpu.SemaphoreType.DMA,
            pltpu.VMEM_SHARED((NPAD, D), jnp.float32),
        ],
    )(_msg_kernel)
    return k(src, dst, h2)


def _bn_body(seg_ref, degT_ref, b_ref, g_ref, be_ref, out_ref):
    seg = seg_ref[0, :N, :] + seg_ref[1, :N, :]
    dsum = degT_ref[:N, 0:1] + degT_ref[:N, 1:2] + 1.0
    dinv = lax.rsqrt(dsum)
    z = dinv * seg + b_ref[...]
    mean = jnp.mean(z, axis=0, keepdims=True)
    zc = z - mean
    var = jnp.mean(zc * zc, axis=0, keepdims=True)
    y = zc * lax.rsqrt(var + 1e-5) * g_ref[...] + be_ref[...]
    out_ref[...] = jnp.maximum(y, 0.0)


def _bn_relu(seg_pair, degT, b, gamma, beta):
    return pl.pallas_call(
        _bn_body,
        out_shape=jax.ShapeDtypeStruct((N, D), jnp.float32),
    )(seg_pair, degT, b, gamma, beta)


def kernel(x, edge_index, W, b, gamma, beta):
    src = edge_index[0]
    dst = edge_index[1]
    deg_pair = _degree_partials(dst)
    degT = jnp.transpose(deg_pair)
    h2 = _h2_compute(x, W, degT)
    seg_pair = _message_partials(src, dst, h2)
    return _bn_relu(seg_pair, degT,
                    jnp.reshape(b, (1, D)),
                    jnp.reshape(gamma, (1, D)),
                    jnp.reshape(beta, (1, D)))


# speedup vs baseline: 1.0384x; 1.0384x over previous
import functools

import jax
import jax.numpy as jnp
from jax import lax
from jax.experimental import pallas as pl
from jax.experimental.pallas import tpu as pltpu
from jax.experimental.pallas import tpu_sc as plsc

N = 10000
E = 320000
D = 128
NPAD = 10240
NC = 2
NS = 16
NW = NC * NS
EPT = E // NW
CH = 80
CPT = EPT // CH
NBUF = 5
NWAVE = CPT // NBUF
RPT = NPAD // NS

CH3 = 40
CPT3 = EPT // CH3
NWAVE3 = CPT3 // NBUF


def _sc_mesh():
    return plsc.VectorSubcoreMesh(core_axis_name="c", subcore_axis_name="s")


def _deg_kernel(dst3_hbm, out_hbm, didx_v, ones_v, bounce_v, ssem, deg_sh):
    c = lax.axis_index("c")
    s = lax.axis_index("s")
    wid = s * NC + c

    for j in range(RPT // 16):
        bounce_v[pl.ds(j * 16, 16)] = jnp.zeros((16,), jnp.float32)
    pltpu.sync_copy(bounce_v, deg_sh.at[pl.ds(s * RPT, RPT)])
    for j in range(CH // 16):
        ones_v[pl.ds(j * 16, 16)] = jnp.ones((16,), jnp.float32)
    plsc.subcore_barrier()

    def wait_sc(j):
        pltpu.make_async_copy(out_hbm.at[0, pl.ds(0, CH)], ones_v, ssem.at[j]).wait()

    pltpu.sync_copy(dst3_hbm.at[wid], didx_v)
    for j in range(NBUF):
        pltpu.async_copy(ones_v, deg_sh.at[didx_v.at[j]], ssem.at[j], add=True)

    def wave(g, carry):
        for j in range(NBUF):
            wait_sc(j)
            pltpu.async_copy(
                ones_v, deg_sh.at[didx_v.at[g * NBUF + j]], ssem.at[j], add=True
            )
        return carry

    lax.fori_loop(1, NWAVE, wave, 0)
    for j in range(NBUF):
        wait_sc(j)
    plsc.subcore_barrier()

    pltpu.sync_copy(deg_sh.at[pl.ds(s * RPT, RPT)], out_hbm.at[c, pl.ds(s * RPT, RPT)])


def _degree_partials(dst3):
    k = functools.partial(
        pl.kernel,
        out_type=jax.ShapeDtypeStruct((NC, NPAD), jnp.float32),
        mesh=_sc_mesh(),
        scratch_types=[
            pltpu.VMEM((CPT, CH), jnp.int32),
            pltpu.VMEM((CH,), jnp.float32),
            pltpu.VMEM((RPT,), jnp.float32),
            pltpu.SemaphoreType.DMA((NBUF,)),
            pltpu.VMEM_SHARED((NPAD,), jnp.float32),
        ],
    )(_deg_kernel)
    return k(dst3)


def _h2_body(x_ref, w_ref, degT_ref, h2_ref):
    dsum = degT_ref[:, 0:1] + degT_ref[:, 1:2] + 1.0
    dinv = lax.rsqrt(dsum)
    h = jnp.dot(x_ref[...], w_ref[...], preferred_element_type=jnp.float32)
    h2_ref[...] = h * dinv


def _h2_compute(x, W, degT):
    nb = 10
    rb = N // nb
    return pl.pallas_call(
        _h2_body,
        grid=(nb,),
        in_specs=[
            pl.BlockSpec((rb, D), lambda i: (i, 0)),
            pl.BlockSpec((D, D), lambda i: (0, 0)),
            pl.BlockSpec((rb, NC), lambda i: (i, 0)),
        ],
        out_specs=pl.BlockSpec((rb, D), lambda i: (i, 0)),
        out_shape=jax.ShapeDtypeStruct((N, D), jnp.float32),
    )(x, W, degT)


def _msg_kernel(src_hbm, dst_hbm, h2_hbm, out_hbm,
                sidx_v, didx_v, rows_v, gsem, ssem, isem, zsem, acc_sh):
    c = lax.axis_index("c")
    s = lax.axis_index("s")
    wid = s * NC + c

    def zrow(i, carry):
        for j in range(D // 16):
            rows_v[0, i, pl.ds(j * 16, 16)] = jnp.zeros((16,), jnp.float32)
        return carry

    lax.fori_loop(0, CH3, zrow, 0)

    def drain_z(k):
        for _ in range(k):
            pltpu.make_async_copy(
                h2_hbm.at[pl.ds(0, CH3)], rows_v.at[0], zsem
            ).wait()

    @pl.when(c == 0)
    def _():
        @pl.when(s < NS - 1)
        def _():
            pltpu.sync_copy(h2_hbm.at[pl.ds(s * RPT, RPT)],
                            acc_sh.at[pl.ds(s * RPT, RPT)])

        @pl.when(s == NS - 1)
        def _():
            pltpu.sync_copy(h2_hbm.at[pl.ds((NS - 1) * RPT, N - (NS - 1) * RPT)],
                            acc_sh.at[pl.ds((NS - 1) * RPT, N - (NS - 1) * RPT)])
            for v in range((NPAD - N) // CH3):
                pltpu.async_copy(rows_v.at[0],
                                 acc_sh.at[pl.ds(N + v * CH3, CH3)], zsem)
            drain_z((NPAD - N) // CH3)

    @pl.when(c == 1)
    def _():
        for v in range(RPT // CH3):
            pltpu.async_copy(rows_v.at[0],
                             acc_sh.at[pl.ds(s * RPT + v * CH3, CH3)], zsem)
        drain_z(RPT // CH3)

    plsc.subcore_barrier()

    pltpu.sync_copy(src_hbm.at[pl.ds(wid * EPT, EPT)], sidx_v)
    base = wid * EPT

    def wait_rows(sem_ref, j):
        pltpu.make_async_copy(h2_hbm.at[pl.ds(0, CH3)], rows_v.at[j], sem_ref.at[j]).wait()

    def wait_idx(j):
        pltpu.make_async_copy(
            dst_hbm.at[pl.ds(0, CH3)], didx_v.at[j], isem.at[j]
        ).wait()

    for j in range(NBUF):
        pltpu.async_copy(dst_hbm.at[pl.ds(base + j * CH3, CH3)],
                         didx_v.at[j], isem.at[j])
        pltpu.async_copy(h2_hbm.at[sidx_v.at[pl.ds(j * CH3, CH3)]],
                         rows_v.at[j], gsem.at[j])

    def wave(g, carry):
        for j in range(NBUF):
            wait_rows(gsem, j)
            wait_idx(j)
            pltpu.async_copy(rows_v.at[j], acc_sh.at[didx_v.at[j]],
                             ssem.at[j], add=True)
        for j in range(NBUF):
            t = (g + 1) * NBUF + j
            wait_rows(ssem, j)
            pltpu.async_copy(dst_hbm.at[pl.ds(base + t * CH3, CH3)],
                             didx_v.at[j], isem.at[j])
            pltpu.async_copy(h2_hbm.at[sidx_v.at[pl.ds(t * CH3, CH3)]],
                             rows_v.at[j], gsem.at[j])
        return carry

    lax.fori_loop(0, NWAVE3 - 1, wave, 0)
    for j in range(NBUF):
        wait_rows(gsem, j)
        wait_idx(j)
        pltpu.async_copy(rows_v.at[j], acc_sh.at[didx_v.at[j]],
                         ssem.at[j], add=True)
    for j in range(NBUF):
        wait_rows(ssem, j)
    plsc.subcore_barrier()

    pltpu.sync_copy(acc_sh.at[pl.ds(s * RPT, RPT)],
                    out_hbm.at[c, pl.ds(s * RPT, RPT)])


def _message_partials(src, dst, h2):
    k = functools.partial(
        pl.kernel,
        out_type=jax.ShapeDtypeStruct((NC, NPAD, D), jnp.float32),
        mesh=_sc_mesh(),
        scratch_types=[
            pltpu.VMEM((EPT,), jnp.int32),
            pltpu.VMEM((NBUF, CH3), jnp.int32),
            pltpu.VMEM((NBUF, CH3, D), jnp.float32),
            pltpu.SemaphoreType.DMA((NBUF,)),
            pltpu.SemaphoreType.DMA((NBUF,)),
            pltpu.SemaphoreType.DMA((NBUF,)),
            pltpu.SemaphoreType.DMA,
            pltpu.VMEM_SHARED((NPAD, D), jnp.float32),
        ],
    )(_msg_kernel)
    return k(src, dst, h2)


def _bn_body(seg_ref, degT_ref, b_ref, g_ref, be_ref, out_ref):
    seg = seg_ref[0, :N, :] + seg_ref[1, :N, :]
    dsum = degT_ref[:N, 0:1] + degT_ref[:N, 1:2] + 1.0
    dinv = lax.rsqrt(dsum)
    z = dinv * seg + b_ref[...]
    mean = jnp.mean(z, axis=0, keepdims=True)
    zc = z - mean
    var = jnp.mean(zc * zc, axis=0, keepdims=True)
    y = zc * lax.rsqrt(var + 1e-5) * g_ref[...] + be_ref[...]
    out_ref[...] = jnp.maximum(y, 0.0)


def _bn_relu(seg_pair, degT, b, gamma, beta):
    return pl.pallas_call(
        _bn_body,
        out_shape=jax.ShapeDtypeStruct((N, D), jnp.float32),
    )(seg_pair, degT, b, gamma, beta)


def kernel(x, edge_index, W, b, gamma, beta):
    src = edge_index[0]
    dst = edge_index[1]
    deg_pair = _degree_partials(
        jnp.reshape(dst, (NW, CPT, CH)))
    degT = jnp.transpose(deg_pair)
    h2 = _h2_compute(x, W, degT)
    seg_pair = _message_partials(src, dst, h2)
    return _bn_relu(seg_pair, degT,
                    jnp.reshape(b, (1, D)),
                    jnp.reshape(gamma, (1, D)),
                    jnp.reshape(beta, (1, D)))
